# Initial kernel scaffold; baseline (speedup 1.0000x reference)
import jax
import jax.numpy as jnp
from jax.experimental import pallas as pl


def kernel(pos, edge_index, batch, W_ds, b_ds, g_ds, be_ds, W_dd, b_dd, g_dd, be_dd, W1, b1, g1, be1, W2, b2, g2, be2, W3, b3, g3, be3, W4, b4):
    raise NotImplementedError("write your pallas kernel here")


# speedup vs baseline: 4.2458x; 4.2458x over previous
import jax
import jax.numpy as jnp
from jax.experimental import pallas as pl

P = 1024
B = 16
K = 20
N = B * P
E = N * K
F_BIG = 1e10
I_BIG = 1 << 30


def _dot(a, b):
    return jax.lax.dot_general(
        a, b, (((1,), (0,)), ((), ())), preferred_element_type=jnp.float32)


def _split3(x):
    h1 = x.astype(jnp.bfloat16).astype(jnp.float32)
    r1 = x - h1
    h2 = r1.astype(jnp.bfloat16).astype(jnp.float32)
    r2 = r1 - h2
    return h1, h2, r2


def _stage1_body(pos_ref, post_ref, wb_ref, bds_ref, idx_ref, f_ref):
    x = pos_ref[0]
    xt = post_ref[0]
    n2c = jnp.sum(x * x, axis=1, keepdims=True)
    n2r = jnp.sum(xt * xt, axis=0, keepdims=True)
    d = (n2c + n2r) - 2.0 * _dot(x.astype(jnp.bfloat16),
                                 xt.astype(jnp.bfloat16))
    row_i = jax.lax.broadcasted_iota(jnp.int32, (P, P), 0)
    col_i = jax.lax.broadcasted_iota(jnp.int32, (P, P), 1)
    d = jnp.where(row_i == col_i, F_BIG, d)

    h1, h2, r2 = _split3(x)
    t3 = jnp.concatenate([h1, h2, r2], axis=1).astype(jnp.bfloat16)
    wb = wb_ref[:]
    bds = bds_ref[:]

    f = jnp.full((P, 60), -jnp.inf, jnp.float32)
    cols = []
    for _ in range(K):
        minv = jnp.min(d, axis=1, keepdims=True)
        cand = jnp.where(d == minv, col_i, I_BIG)
        sel = jnp.min(cand, axis=1, keepdims=True)
        onehot = col_i == sel
        d = jnp.where(onehot, F_BIG, d)
        ohb = jnp.where(onehot, 1.0, 0.0).astype(jnp.bfloat16)
        g = _dot(ohb, t3)
        ps = (g[:, 0:3] + g[:, 3:6]) + g[:, 6:9]
        rb = (ps - x).astype(jnp.bfloat16).astype(jnp.float32)
        hk = ((rb[:, 0:1] * wb[0:1, :] + rb[:, 1:2] * wb[1:2, :])
              + rb[:, 2:3] * wb[2:3, :]) + bds
        f = jnp.maximum(f, jnp.maximum(hk, 0.0))
        cols.append(sel)
    idx_ref[0] = jnp.concatenate(cols, axis=1)
    f_ref[0] = f


def _stage2_body(f_ref, idx_ref, wa_ref, wbb_ref, wc_ref, bdd_ref,
                 f2_ref, zsum_ref, zss_ref):
    fv = f_ref[0]
    s1, s2, s3 = _split3(fv)
    s1b = s1.astype(jnp.bfloat16)
    s2b = s2.astype(jnp.bfloat16)
    s3b = s3.astype(jnp.bfloat16)
    wa = wa_ref[:].astype(jnp.bfloat16)
    wbb = wbb_ref[:].astype(jnp.bfloat16)
    wc = wc_ref[:].astype(jnp.bfloat16)
    bdd = bdd_ref[:]
    pm = _dot(s1b, wa)

    col_i = jax.lax.broadcasted_iota(jnp.int32, (P, P), 1)
    idx = idx_ref[0]
    f2 = jnp.full((P, 128), -jnp.inf, jnp.float32)
    zsum = jnp.zeros((1, 128), jnp.float32)
    zss = jnp.zeros((1, 128), jnp.float32)
    for k in range(K):
        sel = idx[:, k:k + 1]
        onehot = col_i == sel
        ohb = jnp.where(onehot, 1.0, 0.0).astype(jnp.bfloat16)
        g1 = _dot(ohb, s1b)
        g2 = _dot(ohb, s2b)
        g3 = _dot(ohb, s3b)
        fs = (g1 + g2) + g3
        db = (fs - fv).astype(jnp.bfloat16)
        fsb = fs.astype(jnp.bfloat16)
        z = ((pm + _dot(db, wbb)) + _dot(fsb, wc)) + bdd
        z = jnp.maximum(z, 0.0)
        f2 = jnp.maximum(f2, z)
        zsum = zsum + jnp.sum(z, axis=0, keepdims=True)
        zss = zss + jnp.sum(z * z, axis=0, keepdims=True)
    f2_ref[0] = f2
    zsum_ref[0] = zsum
    zss_ref[0] = zss


def _stage3_body(f2_ref, w1_ref, y1_ref):
    y = _dot(f2_ref[0].astype(jnp.bfloat16),
             w1_ref[:].astype(jnp.bfloat16))
    y1_ref[0] = jnp.max(y, axis=0, keepdims=True)


def _bn_rows(x, g, b, eps=1e-5):
    m = jnp.mean(x, axis=0, keepdims=True)
    v = jnp.mean((x - m) ** 2, axis=0, keepdims=True)
    return (x - m) / jnp.sqrt(v + eps) * g + b


def _bdot(a, b):
    return _dot(a.astype(jnp.bfloat16), b.astype(jnp.bfloat16))


def _stage4_body(y_ref, g1_ref, be1_ref, w2_ref, b2_ref, g2_ref, be2_ref,
                 w3_ref, b3_ref, g3_ref, be3_ref, w4_ref, b4_ref, out_ref):
    y = jnp.maximum(y_ref[:], 0.0)
    y = _bn_rows(y, g1_ref[0][None, :], be1_ref[0][None, :])
    y = jnp.maximum(_bdot(y, w2_ref[:]) + b2_ref[0][None, :], 0.0)
    y = _bn_rows(y, g2_ref[0][None, :], be2_ref[0][None, :])
    y = jnp.maximum(_bdot(y, w3_ref[:]) + b3_ref[0][None, :], 0.0)
    y = _bn_rows(y, g3_ref[0][None, :], be3_ref[0][None, :])
    y = _bdot(y, w4_ref[:]) + b4_ref[0][None, :]
    mx = jnp.max(y, axis=1, keepdims=True)
    lse = jnp.log(jnp.sum(jnp.exp(y - mx), axis=1, keepdims=True))
    out_ref[:] = y - mx - lse


def _full(shape):
    nd = len(shape)
    return pl.BlockSpec(shape, lambda b: (0,) * nd)


def _per_cloud(shape):
    nd = len(shape)
    return pl.BlockSpec((1,) + shape[1:], lambda b: (b,) + (0,) * (nd - 1))


@jax.jit
def kernel(pos, edge_index, batch, W_ds, b_ds, g_ds, be_ds,
           W_dd, b_dd, g_dd, be_dd, W1, b1, g1, be1, W2, b2, g2, be2,
           W3, b3, g3, be3, W4, b4):
    del edge_index, batch
    pos_b = pos.reshape(B, P, 3)
    pos_t = jnp.transpose(pos_b, (0, 2, 1))
    wdsb = W_ds.astype(jnp.bfloat16).astype(jnp.float32)

    idx, f = pl.pallas_call(
        _stage1_body,
        grid=(B,),
        in_specs=[_per_cloud((B, P, 3)), _per_cloud((B, 3, P)),
                  _full((3, 60)), _full((1, 60))],
        out_specs=[_per_cloud((B, P, K)), _per_cloud((B, P, 60))],
        out_shape=[
            jax.ShapeDtypeStruct((B, P, K), jnp.int32),
            jax.ShapeDtypeStruct((B, P, 60), jnp.float32),
        ],
    )(pos_b, pos_t, wdsb, b_ds.reshape(1, 60))

    f2d = f.reshape(N, 60)
    fbn = ((f2d - jnp.mean(f2d, axis=0)) /
           jnp.sqrt(jnp.var(f2d, axis=0) + 1e-5) * g_ds + be_ds)

    f2, zsum, zss = pl.pallas_call(
        _stage2_body,
        grid=(B,),
        in_specs=[_per_cloud((B, P, 60)), _per_cloud((B, P, K)),
                  _full((60, 128)), _full((60, 128)), _full((60, 128)),
                  _full((1, 128))],
        out_specs=[_per_cloud((B, P, 128)), _per_cloud((B, 1, 128)),
                   _per_cloud((B, 1, 128))],
        out_shape=[
            jax.ShapeDtypeStruct((B, P, 128), jnp.float32),
            jax.ShapeDtypeStruct((B, 1, 128), jnp.float32),
            jax.ShapeDtypeStruct((B, 1, 128), jnp.float32),
        ],
    )(fbn.reshape(B, P, 60), idx, W_dd[0:60], W_dd[60:120], W_dd[120:180],
      b_dd.reshape(1, 128))

    mean_z = jnp.sum(zsum, axis=(0, 1)) / E
    var_z = jnp.sum(zss, axis=(0, 1)) / E - mean_z * mean_z
    f2bn = ((f2 - mean_z) / jnp.sqrt(var_z + 1e-5) * g_dd + be_dd)

    y1 = pl.pallas_call(
        _stage3_body,
        grid=(B,),
        in_specs=[_per_cloud((B, P, 128)), _full((128, 1024))],
        out_specs=_per_cloud((B, 1, 1024)),
        out_shape=jax.ShapeDtypeStruct((B, 1, 1024), jnp.float32),
    )(f2bn, W1)

    y1b = y1.reshape(B, 1024) + b1[None, :]

    out = pl.pallas_call(
        _stage4_body,
        grid=(1,),
        in_specs=[_full((B, 1024)), _full((1, 1024)), _full((1, 1024)),
                  _full((1024, 512)), _full((1, 512)), _full((1, 512)),
                  _full((1, 512)),
                  _full((512, 265)), _full((1, 265)), _full((1, 265)),
                  _full((1, 265)),
                  _full((265, 40)), _full((1, 40))],
        out_specs=_full((B, 40)),
        out_shape=jax.ShapeDtypeStruct((B, 40), jnp.float32),
    )(y1b, g1.reshape(1, -1), be1.reshape(1, -1),
      W2, b2.reshape(1, -1), g2.reshape(1, -1), be2.reshape(1, -1),
      W3, b3.reshape(1, -1), g3.reshape(1, -1), be3.reshape(1, -1),
      W4, b4.reshape(1, -1))
    return out


# speedup vs baseline: 4.2477x; 1.0005x over previous
import jax
import jax.numpy as jnp
from jax.experimental import pallas as pl
from jax.experimental.pallas import tpu as pltpu

_PAR = pltpu.CompilerParams(dimension_semantics=("parallel",))

P = 1024
B = 16
K = 20
N = B * P
E = N * K
F_BIG = 1e10
I_BIG = 1 << 30


def _dot(a, b):
    return jax.lax.dot_general(
        a, b, (((1,), (0,)), ((), ())), preferred_element_type=jnp.float32)


def _split3(x):
    h1 = x.astype(jnp.bfloat16).astype(jnp.float32)
    r1 = x - h1
    h2 = r1.astype(jnp.bfloat16).astype(jnp.float32)
    r2 = r1 - h2
    return h1, h2, r2


def _stage1_body(pos_ref, post_ref, wb_ref, bds_ref, idx_ref, f_ref):
    x = pos_ref[0]
    xt = post_ref[0]
    n2c = jnp.sum(x * x, axis=1, keepdims=True)
    n2r = jnp.sum(xt * xt, axis=0, keepdims=True)
    d = (n2c + n2r) - 2.0 * _dot(x.astype(jnp.bfloat16),
                                 xt.astype(jnp.bfloat16))
    row_i = jax.lax.broadcasted_iota(jnp.int32, (P, P), 0)
    col_i = jax.lax.broadcasted_iota(jnp.int32, (P, P), 1)
    d = jnp.where(row_i == col_i, F_BIG, d)

    h1, h2, r2 = _split3(x)
    t3 = jnp.concatenate([h1, h2, r2], axis=1).astype(jnp.bfloat16)
    wb = wb_ref[:]
    bds = bds_ref[:]

    f = jnp.full((P, 60), -jnp.inf, jnp.float32)
    cols = []
    for _ in range(K):
        minv = jnp.min(d, axis=1, keepdims=True)
        cand = jnp.where(d == minv, col_i, I_BIG)
        sel = jnp.min(cand, axis=1, keepdims=True)
        onehot = col_i == sel
        d = jnp.where(onehot, F_BIG, d)
        ohb = jnp.where(onehot, 1.0, 0.0).astype(jnp.bfloat16)
        g = _dot(ohb, t3)
        ps = (g[:, 0:3] + g[:, 3:6]) + g[:, 6:9]
        rb = (ps - x).astype(jnp.bfloat16).astype(jnp.float32)
        hk = ((rb[:, 0:1] * wb[0:1, :] + rb[:, 1:2] * wb[1:2, :])
              + rb[:, 2:3] * wb[2:3, :]) + bds
        f = jnp.maximum(f, jnp.maximum(hk, 0.0))
        cols.append(sel)
    idx_ref[0] = jnp.concatenate(cols, axis=1)
    f_ref[0] = f


def _stage2_body(f_ref, idx_ref, wa_ref, wbb_ref, wc_ref, bdd_ref,
                 f2_ref, zsum_ref, zss_ref):
    fv = f_ref[0]
    s1, s2, s3 = _split3(fv)
    s1b = s1.astype(jnp.bfloat16)
    s2b = s2.astype(jnp.bfloat16)
    s3b = s3.astype(jnp.bfloat16)
    wa = wa_ref[:].astype(jnp.bfloat16)
    wbb = wbb_ref[:].astype(jnp.bfloat16)
    wc = wc_ref[:].astype(jnp.bfloat16)
    bdd = bdd_ref[:]
    pm = _dot(s1b, wa)

    col_i = jax.lax.broadcasted_iota(jnp.int32, (P, P), 1)
    idx = idx_ref[0]
    f2 = jnp.full((P, 128), -jnp.inf, jnp.float32)
    zsum = jnp.zeros((1, 128), jnp.float32)
    zss = jnp.zeros((1, 128), jnp.float32)
    for k in range(K):
        sel = idx[:, k:k + 1]
        onehot = col_i == sel
        ohb = jnp.where(onehot, 1.0, 0.0).astype(jnp.bfloat16)
        g1 = _dot(ohb, s1b)
        g2 = _dot(ohb, s2b)
        g3 = _dot(ohb, s3b)
        fs = (g1 + g2) + g3
        db = (fs - fv).astype(jnp.bfloat16)
        fsb = fs.astype(jnp.bfloat16)
        z = ((pm + _dot(db, wbb)) + _dot(fsb, wc)) + bdd
        z = jnp.maximum(z, 0.0)
        f2 = jnp.maximum(f2, z)
        zsum = zsum + jnp.sum(z, axis=0, keepdims=True)
        zss = zss + jnp.sum(z * z, axis=0, keepdims=True)
    f2_ref[0] = f2
    zsum_ref[0] = zsum
    zss_ref[0] = zss


def _stage3_body(f2_ref, w1_ref, y1_ref):
    y = _dot(f2_ref[0].astype(jnp.bfloat16),
             w1_ref[:].astype(jnp.bfloat16))
    y1_ref[0] = jnp.max(y, axis=0, keepdims=True)


def _bn_rows(x, g, b, eps=1e-5):
    m = jnp.mean(x, axis=0, keepdims=True)
    v = jnp.mean((x - m) ** 2, axis=0, keepdims=True)
    return (x - m) / jnp.sqrt(v + eps) * g + b


def _bdot(a, b):
    return _dot(a.astype(jnp.bfloat16), b.astype(jnp.bfloat16))


def _stage4_body(y_ref, g1_ref, be1_ref, w2_ref, b2_ref, g2_ref, be2_ref,
                 w3_ref, b3_ref, g3_ref, be3_ref, w4_ref, b4_ref, out_ref):
    y = jnp.maximum(y_ref[:], 0.0)
    y = _bn_rows(y, g1_ref[0][None, :], be1_ref[0][None, :])
    y = jnp.maximum(_bdot(y, w2_ref[:]) + b2_ref[0][None, :], 0.0)
    y = _bn_rows(y, g2_ref[0][None, :], be2_ref[0][None, :])
    y = jnp.maximum(_bdot(y, w3_ref[:]) + b3_ref[0][None, :], 0.0)
    y = _bn_rows(y, g3_ref[0][None, :], be3_ref[0][None, :])
    y = _bdot(y, w4_ref[:]) + b4_ref[0][None, :]
    mx = jnp.max(y, axis=1, keepdims=True)
    lse = jnp.log(jnp.sum(jnp.exp(y - mx), axis=1, keepdims=True))
    out_ref[:] = y - mx - lse


def _full(shape):
    nd = len(shape)
    return pl.BlockSpec(shape, lambda b: (0,) * nd)


def _per_cloud(shape):
    nd = len(shape)
    return pl.BlockSpec((1,) + shape[1:], lambda b: (b,) + (0,) * (nd - 1))


@jax.jit
def kernel(pos, edge_index, batch, W_ds, b_ds, g_ds, be_ds,
           W_dd, b_dd, g_dd, be_dd, W1, b1, g1, be1, W2, b2, g2, be2,
           W3, b3, g3, be3, W4, b4):
    del edge_index, batch
    pos_b = pos.reshape(B, P, 3)
    pos_t = jnp.transpose(pos_b, (0, 2, 1))
    wdsb = W_ds.astype(jnp.bfloat16).astype(jnp.float32)

    idx, f = pl.pallas_call(
        _stage1_body,
        grid=(B,),
        in_specs=[_per_cloud((B, P, 3)), _per_cloud((B, 3, P)),
                  _full((3, 60)), _full((1, 60))],
        out_specs=[_per_cloud((B, P, K)), _per_cloud((B, P, 60))],
        out_shape=[
            jax.ShapeDtypeStruct((B, P, K), jnp.int32),
            jax.ShapeDtypeStruct((B, P, 60), jnp.float32),
        ],
        compiler_params=_PAR,
    )(pos_b, pos_t, wdsb, b_ds.reshape(1, 60))

    f2d = f.reshape(N, 60)
    fbn = ((f2d - jnp.mean(f2d, axis=0)) /
           jnp.sqrt(jnp.var(f2d, axis=0) + 1e-5) * g_ds + be_ds)

    f2, zsum, zss = pl.pallas_call(
        _stage2_body,
        grid=(B,),
        in_specs=[_per_cloud((B, P, 60)), _per_cloud((B, P, K)),
                  _full((60, 128)), _full((60, 128)), _full((60, 128)),
                  _full((1, 128))],
        out_specs=[_per_cloud((B, P, 128)), _per_cloud((B, 1, 128)),
                   _per_cloud((B, 1, 128))],
        out_shape=[
            jax.ShapeDtypeStruct((B, P, 128), jnp.float32),
            jax.ShapeDtypeStruct((B, 1, 128), jnp.float32),
            jax.ShapeDtypeStruct((B, 1, 128), jnp.float32),
        ],
        compiler_params=_PAR,
    )(fbn.reshape(B, P, 60), idx, W_dd[0:60], W_dd[60:120], W_dd[120:180],
      b_dd.reshape(1, 128))

    mean_z = jnp.sum(zsum, axis=(0, 1)) / E
    var_z = jnp.sum(zss, axis=(0, 1)) / E - mean_z * mean_z
    f2bn = ((f2 - mean_z) / jnp.sqrt(var_z + 1e-5) * g_dd + be_dd)

    y1 = pl.pallas_call(
        _stage3_body,
        grid=(B,),
        in_specs=[_per_cloud((B, P, 128)), _full((128, 1024))],
        out_specs=_per_cloud((B, 1, 1024)),
        out_shape=jax.ShapeDtypeStruct((B, 1, 1024), jnp.float32),
        compiler_params=_PAR,
    )(f2bn, W1)

    y1b = y1.reshape(B, 1024) + b1[None, :]

    out = pl.pallas_call(
        _stage4_body,
        grid=(1,),
        in_specs=[_full((B, 1024)), _full((1, 1024)), _full((1, 1024)),
                  _full((1024, 512)), _full((1, 512)), _full((1, 512)),
                  _full((1, 512)),
                  _full((512, 265)), _full((1, 265)), _full((1, 265)),
                  _full((1, 265)),
                  _full((265, 40)), _full((1, 40))],
        out_specs=_full((B, 40)),
        out_shape=jax.ShapeDtypeStruct((B, 40), jnp.float32),
    )(y1b, g1.reshape(1, -1), be1.reshape(1, -1),
      W2, b2.reshape(1, -1), g2.reshape(1, -1), be2.reshape(1, -1),
      W3, b3.reshape(1, -1), g3.reshape(1, -1), be3.reshape(1, -1),
      W4, b4.reshape(1, -1))
    return out


# speedup vs baseline: 6.0193x; 1.4171x over previous
import jax
import jax.numpy as jnp
from jax.experimental import pallas as pl
from jax.experimental.pallas import tpu as pltpu

_PAR = pltpu.CompilerParams(dimension_semantics=("parallel",))

P = 1024
B = 16
K = 20
N = B * P
E = N * K
F_BIG = 1e10
I_BIG = 1 << 30


def _dot(a, b):
    return jax.lax.dot_general(
        a, b, (((1,), (0,)), ((), ())), preferred_element_type=jnp.float32)


def _split3(x):
    h1 = x.astype(jnp.bfloat16).astype(jnp.float32)
    r1 = x - h1
    h2 = r1.astype(jnp.bfloat16).astype(jnp.float32)
    r2 = r1 - h2
    return h1, h2, r2


def _stage1_body(pos_ref, post_ref, wb_ref, bds_ref, idx_ref, f_ref):
    x = pos_ref[0]
    xt = post_ref[0]
    n2c = jnp.sum(x * x, axis=1, keepdims=True)
    n2r = jnp.sum(xt * xt, axis=0, keepdims=True)
    d = (n2c + n2r) - 2.0 * _dot(x.astype(jnp.bfloat16),
                                 xt.astype(jnp.bfloat16))
    row_i = jax.lax.broadcasted_iota(jnp.int32, (P, P), 0)
    col_i = jax.lax.broadcasted_iota(jnp.int32, (P, P), 1)
    d = jnp.where(row_i == col_i, F_BIG, d)

    h1, h2, r2 = _split3(x)
    t3 = jnp.concatenate([h1, h2, r2], axis=1).astype(jnp.bfloat16)
    wb = wb_ref[:]
    bds = bds_ref[:]

    f = jnp.full((P, 60), -jnp.inf, jnp.float32)
    cols = []
    for _ in range(K):
        minv = jnp.min(d, axis=1, keepdims=True)
        cand = jnp.where(d == minv, col_i, I_BIG)
        sel = jnp.min(cand, axis=1, keepdims=True)
        onehot = col_i == sel
        d = jnp.where(onehot, F_BIG, d)
        ohb = jnp.where(onehot, 1.0, 0.0).astype(jnp.bfloat16)
        g = _dot(ohb, t3)
        ps = (g[:, 0:3] + g[:, 3:6]) + g[:, 6:9]
        rb = (ps - x).astype(jnp.bfloat16).astype(jnp.float32)
        hk = ((rb[:, 0:1] * wb[0:1, :] + rb[:, 1:2] * wb[1:2, :])
              + rb[:, 2:3] * wb[2:3, :]) + bds
        f = jnp.maximum(f, jnp.maximum(hk, 0.0))
        cols.append(sel)
    idx_ref[0] = jnp.concatenate(cols, axis=1)
    f_ref[0] = f


def _stage2_body(f_ref, idx_ref, wa_ref, wbb_ref, wc_ref, bdd_ref,
                 f2_ref, zsum_ref, zss_ref):
    fv = f_ref[0]
    s1, s2, s3 = _split3(fv)
    s1b = s1.astype(jnp.bfloat16)
    t3 = jnp.concatenate([s1, s2, s3], axis=1).astype(jnp.bfloat16)
    wa = wa_ref[:].astype(jnp.bfloat16)
    wbb = wbb_ref[:].astype(jnp.bfloat16)
    wc = wc_ref[:].astype(jnp.bfloat16)
    bdd = bdd_ref[:]
    pm = _dot(s1b, wa)

    col_i = jax.lax.broadcasted_iota(jnp.int32, (P, P), 1)
    idx = idx_ref[0]
    f2 = jnp.full((P, 128), -jnp.inf, jnp.float32)
    zsum = jnp.zeros((1, 128), jnp.float32)
    zss = jnp.zeros((1, 128), jnp.float32)
    for k in range(K):
        sel = idx[:, k:k + 1]
        onehot = col_i == sel
        ohb = jnp.where(onehot, 1.0, 0.0).astype(jnp.bfloat16)
        g = _dot(ohb, t3)
        fs = (g[:, 0:60] + g[:, 60:120]) + g[:, 120:180]
        db = (fs - fv).astype(jnp.bfloat16)
        fsb = fs.astype(jnp.bfloat16)
        z = ((pm + _dot(db, wbb)) + _dot(fsb, wc)) + bdd
        z = jnp.maximum(z, 0.0)
        f2 = jnp.maximum(f2, z)
        zsum = zsum + jnp.sum(z, axis=0, keepdims=True)
        zss = zss + jnp.sum(z * z, axis=0, keepdims=True)
    f2_ref[0] = f2
    zsum_ref[0] = zsum
    zss_ref[0] = zss


def _stage3_body(f2_ref, w1_ref, y1_ref):
    y = _dot(f2_ref[0].astype(jnp.bfloat16),
             w1_ref[:].astype(jnp.bfloat16))
    y1_ref[0] = jnp.max(y, axis=0, keepdims=True)


def _bn_rows(x, g, b, eps=1e-5):
    m = jnp.mean(x, axis=0, keepdims=True)
    v = jnp.mean((x - m) ** 2, axis=0, keepdims=True)
    return (x - m) / jnp.sqrt(v + eps) * g + b


def _bdot(a, b):
    return _dot(a.astype(jnp.bfloat16), b.astype(jnp.bfloat16))


def _stage4_body(y_ref, g1_ref, be1_ref, w2_ref, b2_ref, g2_ref, be2_ref,
                 w3_ref, b3_ref, g3_ref, be3_ref, w4_ref, b4_ref, out_ref):
    y = jnp.maximum(y_ref[:], 0.0)
    y = _bn_rows(y, g1_ref[0][None, :], be1_ref[0][None, :])
    y = jnp.maximum(_bdot(y, w2_ref[:]) + b2_ref[0][None, :], 0.0)
    y = _bn_rows(y, g2_ref[0][None, :], be2_ref[0][None, :])
    y = jnp.maximum(_bdot(y, w3_ref[:]) + b3_ref[0][None, :], 0.0)
    y = _bn_rows(y, g3_ref[0][None, :], be3_ref[0][None, :])
    y = _bdot(y, w4_ref[:]) + b4_ref[0][None, :]
    mx = jnp.max(y, axis=1, keepdims=True)
    lse = jnp.log(jnp.sum(jnp.exp(y - mx), axis=1, keepdims=True))
    out_ref[:] = y - mx - lse


def _full(shape):
    nd = len(shape)
    return pl.BlockSpec(shape, lambda b: (0,) * nd)


def _per_cloud(shape):
    nd = len(shape)
    return pl.BlockSpec((1,) + shape[1:], lambda b: (b,) + (0,) * (nd - 1))


@jax.jit
def kernel(pos, edge_index, batch, W_ds, b_ds, g_ds, be_ds,
           W_dd, b_dd, g_dd, be_dd, W1, b1, g1, be1, W2, b2, g2, be2,
           W3, b3, g3, be3, W4, b4):
    del edge_index, batch
    pos_b = pos.reshape(B, P, 3)
    pos_t = jnp.transpose(pos_b, (0, 2, 1))
    wdsb = W_ds.astype(jnp.bfloat16).astype(jnp.float32)

    idx, f = pl.pallas_call(
        _stage1_body,
        grid=(B,),
        in_specs=[_per_cloud((B, P, 3)), _per_cloud((B, 3, P)),
                  _full((3, 60)), _full((1, 60))],
        out_specs=[_per_cloud((B, P, K)), _per_cloud((B, P, 60))],
        out_shape=[
            jax.ShapeDtypeStruct((B, P, K), jnp.int32),
            jax.ShapeDtypeStruct((B, P, 60), jnp.float32),
        ],
        compiler_params=_PAR,
    )(pos_b, pos_t, wdsb, b_ds.reshape(1, 60))

    f2d = f.reshape(N, 60)
    fbn = ((f2d - jnp.mean(f2d, axis=0)) /
           jnp.sqrt(jnp.var(f2d, axis=0) + 1e-5) * g_ds + be_ds)

    f2, zsum, zss = pl.pallas_call(
        _stage2_body,
        grid=(B,),
        in_specs=[_per_cloud((B, P, 60)), _per_cloud((B, P, K)),
                  _full((60, 128)), _full((60, 128)), _full((60, 128)),
                  _full((1, 128))],
        out_specs=[_per_cloud((B, P, 128)), _per_cloud((B, 1, 128)),
                   _per_cloud((B, 1, 128))],
        out_shape=[
            jax.ShapeDtypeStruct((B, P, 128), jnp.float32),
            jax.ShapeDtypeStruct((B, 1, 128), jnp.float32),
            jax.ShapeDtypeStruct((B, 1, 128), jnp.float32),
        ],
        compiler_params=_PAR,
    )(fbn.reshape(B, P, 60), idx, W_dd[0:60], W_dd[60:120], W_dd[120:180],
      b_dd.reshape(1, 128))

    mean_z = jnp.sum(zsum, axis=(0, 1)) / E
    var_z = jnp.sum(zss, axis=(0, 1)) / E - mean_z * mean_z
    f2bn = ((f2 - mean_z) / jnp.sqrt(var_z + 1e-5) * g_dd + be_dd)

    y1 = pl.pallas_call(
        _stage3_body,
        grid=(B,),
        in_specs=[_per_cloud((B, P, 128)), _full((128, 1024))],
        out_specs=_per_cloud((B, 1, 1024)),
        out_shape=jax.ShapeDtypeStruct((B, 1, 1024), jnp.float32),
        compiler_params=_PAR,
    )(f2bn, W1)

    y1b = y1.reshape(B, 1024) + b1[None, :]

    out = pl.pallas_call(
        _stage4_body,
        grid=(1,),
        in_specs=[_full((B, 1024)), _full((1, 1024)), _full((1, 1024)),
                  _full((1024, 512)), _full((1, 512)), _full((1, 512)),
                  _full((1, 512)),
                  _full((512, 265)), _full((1, 265)), _full((1, 265)),
                  _full((1, 265)),
                  _full((265, 40)), _full((1, 40))],
        out_specs=_full((B, 40)),
        out_shape=jax.ShapeDtypeStruct((B, 40), jnp.float32),
    )(y1b, g1.reshape(1, -1), be1.reshape(1, -1),
      W2, b2.reshape(1, -1), g2.reshape(1, -1), be2.reshape(1, -1),
      W3, b3.reshape(1, -1), g3.reshape(1, -1), be3.reshape(1, -1),
      W4, b4.reshape(1, -1))
    return out
